# Initial kernel scaffold; baseline (speedup 1.0000x reference)
import jax
import jax.numpy as jnp
from jax.experimental import pallas as pl


def kernel(X_image, input_ids, attention_mask, input_ids2, attention_mask2, label, cnn_w, bert_emb, cnn_scale, cnn_shift, bert_scale, bert_shift, w_img, w_t1, w_t2, b_fold, arc_wnt_pad):
    raise NotImplementedError("write your pallas kernel here")


# speedup vs baseline: 1.0004x; 1.0004x over previous
import functools
import math

import jax
import jax.numpy as jnp
from jax.experimental import pallas as pl
from jax.experimental.pallas import tpu as pltpu

_S = 32.0
_M = 0.5
_COS_M = math.cos(_M)
_SIN_M = math.sin(_M)
_TH = math.cos(math.pi - _M)
_MM = math.sin(math.pi - _M) * _M
_NORM_EPS = 1e-12


def _head_body(label_ref, x_ref, t1_ref, t2_ref,
               cs_ref, ch_ref, bs_ref, bh_ref,
               wi_ref, w1_ref, w2_ref, b_ref, wnt_ref,
               logits_ref, ret_ref, fn_ref, *, tn):
    ci = pl.program_id(1)

    @pl.when(ci == 0)
    def _embed():
        xf = x_ref[...] * cs_ref[...] + ch_ref[...]
        t1 = t1_ref[...] * bs_ref[...] + bh_ref[...]
        t2 = t2_ref[...] * bs_ref[...] + bh_ref[...]
        acc = jnp.dot(xf.astype(jnp.bfloat16), wi_ref[...],
                      preferred_element_type=jnp.float32)
        acc = acc + jnp.dot(t1.astype(jnp.bfloat16), w1_ref[...],
                            preferred_element_type=jnp.float32)
        acc = acc + jnp.dot(t2.astype(jnp.bfloat16), w2_ref[...],
                            preferred_element_type=jnp.float32)
        acc = acc + b_ref[...]
        ret_ref[...] = acc
        inv = jax.lax.rsqrt(jnp.sum(acc * acc, axis=1, keepdims=True) + _NORM_EPS)
        fn_ref[...] = (acc * inv).astype(jnp.bfloat16)

    cos = jnp.dot(fn_ref[...], wnt_ref[...], preferred_element_type=jnp.float32)
    sin = jnp.sqrt(jnp.clip(1.0 - cos * cos, 0.0, 1.0))
    phi = jnp.where(cos > _TH, cos * _COS_M - sin * _SIN_M, cos - _MM)
    cls = ci * tn + jax.lax.broadcasted_iota(jnp.int32, cos.shape, 1)
    logits_ref[...] = jnp.where(cls == label_ref[...], phi, cos) * _S


def kernel(X_image, input_ids, attention_mask, input_ids2, attention_mask2,
           label, cnn_w, bert_emb, cnn_scale, cnn_shift, bert_scale,
           bert_shift, w_img, w_t1, w_t2, b_fold, arc_wnt_pad):
    del attention_mask, attention_mask2

    B = X_image.shape[0]
    cf = cnn_w.shape[1]
    hs = bert_emb.shape[1]
    o = b_fold.shape[1]
    C = arc_wnt_pad.shape[1]

    gap = jnp.mean(X_image, axis=(2, 3))
    x = gap @ cnn_w
    t1 = jnp.mean(jnp.take(bert_emb, input_ids, axis=0), axis=1)
    t2 = jnp.mean(jnp.take(bert_emb, input_ids2, axis=0), axis=1)

    tm = 128
    nb = B // tm
    tn = 1024
    nc = C // tn

    label_col = label.astype(jnp.int32).reshape(B, 1)

    blk_b = lambda bi, ci: (bi, 0)
    blk_0 = lambda bi, ci: (0, 0)
    blk_c = lambda bi, ci: (0, ci)
    blk_bc = lambda bi, ci: (bi, ci)

    in_specs = [
        pl.BlockSpec((tm, 1), blk_b),
        pl.BlockSpec((tm, cf), blk_b),
        pl.BlockSpec((tm, hs), blk_b),
        pl.BlockSpec((tm, hs), blk_b),
        pl.BlockSpec((1, cf), blk_0),
        pl.BlockSpec((1, cf), blk_0),
        pl.BlockSpec((1, hs), blk_0),
        pl.BlockSpec((1, hs), blk_0),
        pl.BlockSpec((cf, o), blk_0),
        pl.BlockSpec((hs, o), blk_0),
        pl.BlockSpec((hs, o), blk_0),
        pl.BlockSpec((1, o), blk_0),
        pl.BlockSpec((o, tn), blk_c),
    ]
    out_specs = (
        pl.BlockSpec((tm, tn), blk_bc),
        pl.BlockSpec((tm, o), blk_b),
    )
    logits, ret = pl.pallas_call(
        functools.partial(_head_body, tn=tn),
        grid=(nb, nc),
        out_shape=(jax.ShapeDtypeStruct((B, C), jnp.float32),
                   jax.ShapeDtypeStruct((B, o), jnp.float32)),
        in_specs=in_specs,
        out_specs=out_specs,
        scratch_shapes=[pltpu.VMEM((tm, o), jnp.bfloat16)],
        compiler_params=pltpu.CompilerParams(
            dimension_semantics=("parallel", "arbitrary"),
            vmem_limit_bytes=48 * 1024 * 1024),
    )(label_col, x, t1, t2, cnn_scale, cnn_shift, bert_scale, bert_shift,
      w_img, w_t1, w_t2, b_fold, arc_wnt_pad)
    return logits, ret
